# baseline (speedup 1.0000x reference)
import jax
import jax.numpy as jnp
from jax import lax
from jax.experimental import pallas as pl
from jax.experimental.pallas import tpu as pltpu
from jax.experimental.pallas import tpu_sc as plsc

B = 1024
C = 3
S = 12
P = 32
D = P * P
SD = S * D

NC = 2
NS = 16
NW = NC * NS
BPW = B // NW


def _sc_body(x_hbm, idx_hbm, table_hbm, out_hbm, idx_v, rows_v, sem_g, sem_x):
    wid = lax.axis_index("s") * NC + lax.axis_index("c")
    base = wid * BPW
    cp_x = pltpu.make_async_copy(
        x_hbm.at[pl.ds(base, BPW)],
        out_hbm.at[pl.ds(base, BPW), pl.ds(0, C)],
        sem_x,
    )
    cp_x.start()
    pltpu.sync_copy(idx_hbm.at[pl.ds(base, BPW)], idx_v)
    pltpu.async_copy(table_hbm.at[idx_v], rows_v, sem_g).wait()
    for s in range(S):
        pltpu.sync_copy(rows_v, out_hbm.at[pl.ds(base, BPW), C, pl.ds(s * D, D)])
    cp_x.wait()


def kernel(x, id, table):
    x3 = x.reshape(B, C, SD)
    out3 = pl.kernel(
        _sc_body,
        out_type=jax.ShapeDtypeStruct((B, C + 1, SD), jnp.float32),
        mesh=plsc.VectorSubcoreMesh(core_axis_name="c", subcore_axis_name="s"),
        scratch_types=[
            pltpu.VMEM((BPW,), jnp.int32),
            pltpu.VMEM((BPW, D), jnp.float32),
            pltpu.SemaphoreType.DMA,
            pltpu.SemaphoreType.DMA,
        ],
    )(x3, id, table)
    return out3.reshape(B, C + 1, S, P, P)
